# Initial kernel scaffold; baseline (speedup 1.0000x reference)
import jax
import jax.numpy as jnp
from jax.experimental import pallas as pl


def kernel(x, Wq, bq, Wk, bk, Wv, bv, Wbeta, bbeta, Wo, bo):
    raise NotImplementedError("write your pallas kernel here")


# speedup vs baseline: 34.8858x; 34.8858x over previous
import jax
import jax.numpy as jnp
from jax.experimental import pallas as pl
from jax.experimental.pallas import tpu as pltpu

_C = 128
_PREC = jax.lax.Precision.HIGHEST


def _mm(a, b):
    return jax.lax.dot_general(a, b, (((1,), (0,)), ((), ())),
                               precision=_PREC,
                               preferred_element_type=jnp.float32)


def _mmt(a, b):
    return jax.lax.dot_general(a, b, (((1,), (1,)), ((), ())),
                               precision=_PREC,
                               preferred_element_type=jnp.float32)


def _mtm(a, b):
    return jax.lax.dot_general(a, b, (((0,), (0,)), ((), ())),
                               precision=_PREC,
                               preferred_element_type=jnp.float32)


def _dn_body(x_ref, wqt_ref, wkt_ref, wvt_ref, wot_ref,
             bq_ref, bk_ref, bv_ref, bo_ref, wbeta_ref, bbeta_ref,
             o_ref, s_ref):
    C = x_ref.shape[1]
    ti = pl.program_id(1)

    @pl.when(ti == 0)
    def _():
        s_ref[...] = jnp.zeros_like(s_ref)

    xc = x_ref[0]
    q = _mm(xc, wqt_ref[...]) + bq_ref[...]
    k = _mm(xc, wkt_ref[...]) + bk_ref[...]
    v = _mm(xc, wvt_ref[...]) + bv_ref[...]
    nrm = jnp.sqrt(jnp.sum(k * k, axis=-1, keepdims=True))
    k = k / jnp.maximum(nrm, 1e-12)
    beta = jax.nn.sigmoid(
        jnp.sum(k * wbeta_ref[...], axis=-1, keepdims=True)
        + bbeta_ref[0, 0])

    row = jax.lax.broadcasted_iota(jnp.int32, (C, C), 0)
    col = jax.lax.broadcasted_iota(jnp.int32, (C, C), 1)
    a = jnp.where(row > col, beta * _mmt(k, k), 0.0)
    eye = jnp.where(row == col, 1.0, 0.0)
    x_inv = eye - a
    for _ in range(max((C - 1).bit_length() - 1, 0)):
        y = x_inv + _mm(a, x_inv)
        x_inv = 2.0 * x_inv - _mm(x_inv, y)

    st = s_ref[...]
    kst = _mm(k, st)
    u = _mm(x_inv, beta * (v - kst))
    qk = jnp.where(row >= col, _mmt(q, k), 0.0)
    o = _mm(q, st) + _mm(qk, u)
    s_ref[...] = st + _mtm(k, u)
    o_ref[0] = _mm(o, wot_ref[...]) + bo_ref[...]


def kernel(x, Wq, bq, Wk, bk, Wv, bv, Wbeta, bbeta, Wo, bo):
    B, T, D = x.shape
    C = _C
    assert T % C == 0
    full = lambda b, t: (0, 0)
    return pl.pallas_call(
        _dn_body,
        out_shape=jax.ShapeDtypeStruct((B, T, D), x.dtype),
        grid=(B, T // C),
        in_specs=[
            pl.BlockSpec((1, C, D), lambda b, t: (b, t, 0)),
            pl.BlockSpec((D, D), full),
            pl.BlockSpec((D, D), full),
            pl.BlockSpec((D, D), full),
            pl.BlockSpec((D, D), full),
            pl.BlockSpec((1, D), full),
            pl.BlockSpec((1, D), full),
            pl.BlockSpec((1, D), full),
            pl.BlockSpec((1, D), full),
            pl.BlockSpec((1, D), full),
            pl.BlockSpec((1, 1), full),
        ],
        out_specs=pl.BlockSpec((1, C, D), lambda b, t: (b, t, 0)),
        scratch_shapes=[pltpu.VMEM((D, D), jnp.float32)],
        compiler_params=pltpu.CompilerParams(
            dimension_semantics=("parallel", "arbitrary"),
            vmem_limit_bytes=56 * 1024 * 1024,
        ),
        name="deltanet_chunked",
        interpret=False,
    )(x, Wq.T, Wk.T, Wv.T, Wo.T,
      bq.reshape(1, D), bk.reshape(1, D), bv.reshape(1, D),
      bo.reshape(1, D), Wbeta.reshape(1, D), bbeta.reshape(1, 1))


# speedup vs baseline: 63.1157x; 1.8092x over previous
import jax
import jax.numpy as jnp
from jax.experimental import pallas as pl
from jax.experimental.pallas import tpu as pltpu

_C = 128

_MM = (((1,), (0,)), ((), ()))
_MT = (((1,), (1,)), ((), ()))
_TM = (((0,), (0,)), ((), ()))


def _d(a, b, dims):
    return jax.lax.dot_general(a, b, dims,
                               preferred_element_type=jnp.float32)


def _split(x):
    hi = x.astype(jnp.bfloat16)
    lo = (x - hi.astype(jnp.float32)).astype(jnp.bfloat16)
    return hi, lo


def _mm3(ap, bp, dims):
    ahi, alo = ap
    bhi, blo = bp
    return (_d(ahi, bhi, dims) + (_d(ahi, blo, dims) + _d(alo, bhi, dims)))


def _dn_body(x_ref, wqh_ref, wql_ref, wkh_ref, wkl_ref, wvh_ref, wvl_ref,
             woh_ref, wol_ref, bq_ref, bk_ref, bv_ref, bo_ref,
             wbeta_ref, bbeta_ref, o_ref, s_ref):
    C = x_ref.shape[1]
    ti = pl.program_id(1)

    @pl.when(ti == 0)
    def _():
        s_ref[...] = jnp.zeros_like(s_ref)

    xp = _split(x_ref[0])
    q = _mm3(xp, (wqh_ref[...], wql_ref[...]), _MM) + bq_ref[...]
    k = _mm3(xp, (wkh_ref[...], wkl_ref[...]), _MM) + bk_ref[...]
    v = _mm3(xp, (wvh_ref[...], wvl_ref[...]), _MM) + bv_ref[...]
    nrm = jnp.sqrt(jnp.sum(k * k, axis=-1, keepdims=True))
    k = k / jnp.maximum(nrm, 1e-12)
    beta = jax.nn.sigmoid(
        jnp.sum(k * wbeta_ref[...], axis=-1, keepdims=True)
        + bbeta_ref[0, 0])

    kp = _split(k)
    row = jax.lax.broadcasted_iota(jnp.int32, (C, C), 0)
    col = jax.lax.broadcasted_iota(jnp.int32, (C, C), 1)
    a = jnp.where(row > col, beta * _mm3(kp, kp, _MT), 0.0)
    ap = _split(a)
    eye = jnp.where(row == col, 1.0, 0.0)
    x_inv = eye - a
    for _ in range(max((C - 1).bit_length() - 2, 0)):
        xb = x_inv.astype(jnp.bfloat16)
        y = (x_inv + _d(ap[0], xb, _MM)).astype(jnp.bfloat16)
        x_inv = 2.0 * x_inv - _d(xb, y, _MM)
    xip = _split(x_inv)
    y = x_inv + _mm3(ap, xip, _MM)
    x_inv = 2.0 * x_inv - _mm3(xip, _split(y), _MM)
    mp = _split(x_inv)

    st = s_ref[...]
    stp = _split(st)
    kst = _mm3(kp, stp, _MM)
    u = _mm3(mp, _split(beta * (v - kst)), _MM)
    up = _split(u)
    qp = _split(q)
    qk = jnp.where(row >= col, _mm3(qp, kp, _MT), 0.0)
    o = _mm3(qp, stp, _MM) + _mm3(_split(qk), up, _MM)
    s_ref[...] = st + _mm3(kp, up, _TM)
    o_ref[0] = _mm3(_split(o), (woh_ref[...], wol_ref[...]), _MM) + bo_ref[...]


def kernel(x, Wq, bq, Wk, bk, Wv, bv, Wbeta, bbeta, Wo, bo):
    B, T, D = x.shape
    C = _C
    assert T % C == 0
    full = lambda b, t: (0, 0)
    wspec = pl.BlockSpec((D, D), full)
    bspec = pl.BlockSpec((1, D), full)
    wqh, wql = _split(Wq.T)
    wkh, wkl = _split(Wk.T)
    wvh, wvl = _split(Wv.T)
    woh, wol = _split(Wo.T)
    return pl.pallas_call(
        _dn_body,
        out_shape=jax.ShapeDtypeStruct((B, T, D), x.dtype),
        grid=(B, T // C),
        in_specs=[
            pl.BlockSpec((1, C, D), lambda b, t: (b, t, 0)),
            wspec, wspec, wspec, wspec, wspec, wspec, wspec, wspec,
            bspec, bspec, bspec, bspec, bspec,
            pl.BlockSpec((1, 1), full),
        ],
        out_specs=pl.BlockSpec((1, C, D), lambda b, t: (b, t, 0)),
        scratch_shapes=[pltpu.VMEM((D, D), jnp.float32)],
        compiler_params=pltpu.CompilerParams(
            dimension_semantics=("parallel", "arbitrary"),
            vmem_limit_bytes=56 * 1024 * 1024,
        ),
        name="deltanet_chunked",
        interpret=False,
    )(x, wqh, wql, wkh, wkl, wvh, wvl, woh, wol,
      bq.reshape(1, D), bk.reshape(1, D), bv.reshape(1, D),
      bo.reshape(1, D), Wbeta.reshape(1, D), bbeta.reshape(1, 1))


# speedup vs baseline: 65.5163x; 1.0380x over previous
import jax
import jax.numpy as jnp
from jax.experimental import pallas as pl
from jax.experimental.pallas import tpu as pltpu

_C = 128
_G = 4

_MM = (((1,), (0,)), ((), ()))
_MT = (((1,), (1,)), ((), ()))
_TM = (((0,), (0,)), ((), ()))


def _d(a, b, dims):
    return jax.lax.dot_general(a, b, dims,
                               preferred_element_type=jnp.float32)


def _split(x):
    hi = x.astype(jnp.bfloat16)
    lo = (x - hi.astype(jnp.float32)).astype(jnp.bfloat16)
    return hi, lo


def _mm3(ap, bp, dims):
    ahi, alo = ap
    bhi, blo = bp
    return (_d(ahi, bhi, dims) + (_d(ahi, blo, dims) + _d(alo, bhi, dims)))


def _dn_body(x_ref, wqh_ref, wql_ref, wkh_ref, wkl_ref, wvh_ref, wvl_ref,
             woh_ref, wol_ref, bq_ref, bk_ref, bv_ref, bo_ref,
             wbeta_ref, bbeta_ref, o_ref, s_ref):
    G, C, D = x_ref.shape
    ti = pl.program_id(1)

    @pl.when(ti == 0)
    def _():
        s_ref[...] = jnp.zeros_like(s_ref)

    xp = _split(x_ref[...].reshape(G * C, D))
    q_all = _mm3(xp, (wqh_ref[...], wql_ref[...]), _MM) + bq_ref[...]
    k_all = _mm3(xp, (wkh_ref[...], wkl_ref[...]), _MM) + bk_ref[...]
    v_all = _mm3(xp, (wvh_ref[...], wvl_ref[...]), _MM) + bv_ref[...]
    nrm = jnp.sqrt(jnp.sum(k_all * k_all, axis=-1, keepdims=True))
    k_all = k_all / jnp.maximum(nrm, 1e-12)
    beta_all = jax.nn.sigmoid(
        jnp.sum(k_all * wbeta_ref[...], axis=-1, keepdims=True)
        + bbeta_ref[0, 0])

    row = jax.lax.broadcasted_iota(jnp.int32, (C, C), 0)
    col = jax.lax.broadcasted_iota(jnp.int32, (C, C), 1)
    eye = jnp.where(row == col, 1.0, 0.0)
    n_newton = max((C - 1).bit_length() - 2, 0)

    o_acc = []
    for g in range(G):
        sl = slice(g * C, (g + 1) * C)
        q, k, v, beta = q_all[sl], k_all[sl], v_all[sl], beta_all[sl]
        kp = _split(k)
        a = jnp.where(row > col, beta * _mm3(kp, kp, _MT), 0.0)
        ap = _split(a)
        x_inv = eye - a
        for _ in range(n_newton):
            xb = x_inv.astype(jnp.bfloat16)
            y = (x_inv + _d(ap[0], xb, _MM)).astype(jnp.bfloat16)
            x_inv = 2.0 * x_inv - _d(xb, y, _MM)
        xip = _split(x_inv)
        y = x_inv + _mm3(ap, xip, _MM)
        x_inv = 2.0 * x_inv - _mm3(xip, _split(y), _MM)
        mp = _split(x_inv)

        st = s_ref[g]
        stp = _split(st)
        kst = _mm3(kp, stp, _MM)
        u = _mm3(mp, _split(beta * (v - kst)), _MM)
        up = _split(u)
        qp = _split(q)
        qk = jnp.where(row >= col, _mm3(qp, kp, _MT), 0.0)
        o = _mm3(qp, stp, _MM) + _mm3(_split(qk), up, _MM)
        s_ref[g] = st + _mm3(kp, up, _TM)
        o_acc.append(o)

    o_all = jnp.concatenate(o_acc, axis=0)
    proj = _mm3(_split(o_all), (woh_ref[...], wol_ref[...]), _MM)
    o_ref[...] = (proj + bo_ref[...]).reshape(G, C, D)


def kernel(x, Wq, bq, Wk, bk, Wv, bv, Wbeta, bbeta, Wo, bo):
    B, T, D = x.shape
    C = _C
    G = _G
    assert T % C == 0 and B % G == 0
    full = lambda b, t: (0, 0)
    wspec = pl.BlockSpec((D, D), full)
    bspec = pl.BlockSpec((1, D), full)
    wqh, wql = _split(Wq.T)
    wkh, wkl = _split(Wk.T)
    wvh, wvl = _split(Wv.T)
    woh, wol = _split(Wo.T)
    return pl.pallas_call(
        _dn_body,
        out_shape=jax.ShapeDtypeStruct((B, T, D), x.dtype),
        grid=(B // G, T // C),
        in_specs=[
            pl.BlockSpec((G, C, D), lambda b, t: (b, t, 0)),
            wspec, wspec, wspec, wspec, wspec, wspec, wspec, wspec,
            bspec, bspec, bspec, bspec, bspec,
            pl.BlockSpec((1, 1), full),
        ],
        out_specs=pl.BlockSpec((G, C, D), lambda b, t: (b, t, 0)),
        scratch_shapes=[pltpu.VMEM((G, D, D), jnp.float32)],
        compiler_params=pltpu.CompilerParams(
            dimension_semantics=("parallel", "arbitrary"),
            vmem_limit_bytes=56 * 1024 * 1024,
        ),
        name="deltanet_chunked",
        interpret=False,
    )(x, wqh, wql, wkh, wkl, wvh, wvl, woh, wol,
      bq.reshape(1, D), bk.reshape(1, D), bv.reshape(1, D),
      bo.reshape(1, D), Wbeta.reshape(1, D), bbeta.reshape(1, 1))


# speedup vs baseline: 99.3363x; 1.5162x over previous
import jax
import jax.numpy as jnp
from jax.experimental import pallas as pl
from jax.experimental.pallas import tpu as pltpu

_C = 128
_G = 4

_MM = (((1,), (0,)), ((), ()))
_MT = (((1,), (1,)), ((), ()))
_TM = (((0,), (0,)), ((), ()))


def _d(a, b, dims):
    return jax.lax.dot_general(a, b, dims,
                               preferred_element_type=jnp.float32)


def _split(x):
    hi = x.astype(jnp.bfloat16)
    lo = (x - hi.astype(jnp.float32)).astype(jnp.bfloat16)
    return hi, lo


def _mm3(ap, bp, dims):
    ahi, alo = ap
    bhi, blo = bp
    return (_d(ahi, bhi, dims) + (_d(ahi, blo, dims) + _d(alo, bhi, dims)))


def _dn_body(x_ref, wqh_ref, wql_ref, wkh_ref, wkl_ref, wvh_ref, wvl_ref,
             woh_ref, wol_ref, bq_ref, bk_ref, bv_ref, bo_ref,
             wbeta_ref, bbeta_ref, o_ref, s_ref):
    G, C, D = x_ref.shape
    ti = pl.program_id(1)

    @pl.when(ti == 0)
    def _():
        s_ref[...] = jnp.zeros_like(s_ref)

    xp = _split(x_ref[...].reshape(G * C, D))
    q_all = _mm3(xp, (wqh_ref[...], wql_ref[...]), _MM) + bq_ref[...]
    k_all = _mm3(xp, (wkh_ref[...], wkl_ref[...]), _MM) + bk_ref[...]
    v_all = _mm3(xp, (wvh_ref[...], wvl_ref[...]), _MM) + bv_ref[...]
    nrm = jnp.sqrt(jnp.sum(k_all * k_all, axis=-1, keepdims=True))
    k_all = k_all / jnp.maximum(nrm, 1e-12)
    beta_all = jax.nn.sigmoid(
        jnp.sum(k_all * wbeta_ref[...], axis=-1, keepdims=True)
        + bbeta_ref[0, 0])

    row = jax.lax.broadcasted_iota(jnp.int32, (C, C), 0)
    col = jax.lax.broadcasted_iota(jnp.int32, (C, C), 1)
    eye = jnp.where(row == col, 1.0, 0.0)
    n_newton = max((C - 1).bit_length() - 2, 0)

    gs = range(G)
    sl = [slice(g * C, (g + 1) * C) for g in gs]
    beta = [beta_all[sl[g]] for g in gs]
    kp = [_split(k_all[sl[g]]) for g in gs]
    a = [jnp.where(row > col, beta[g] * _mm3(kp[g], kp[g], _MT), 0.0)
         for g in gs]
    ap = [_split(a[g]) for g in gs]
    x_inv = [eye - a[g] for g in gs]
    for _ in range(n_newton):
        xb = [x_inv[g].astype(jnp.bfloat16) for g in gs]
        y = [(x_inv[g] + _d(ap[g][0], xb[g], _MM)).astype(jnp.bfloat16)
             for g in gs]
        x_inv = [2.0 * x_inv[g] - _d(xb[g], y[g], _MM) for g in gs]
    xip = [_split(x_inv[g]) for g in gs]
    y = [x_inv[g] + _mm3(ap[g], xip[g], _MM) for g in gs]
    mp = [_split(2.0 * x_inv[g] - _mm3(xip[g], _split(y[g]), _MM))
          for g in gs]

    st = [s_ref[g] for g in gs]
    stp = [_split(st[g]) for g in gs]
    kst = [_mm3(kp[g], stp[g], _MM) for g in gs]
    u = [_mm3(mp[g], _split(beta[g] * (v_all[sl[g]] - kst[g])), _MM)
         for g in gs]
    up = [_split(u[g]) for g in gs]
    qp = [_split(q_all[sl[g]]) for g in gs]
    qk = [jnp.where(row >= col, _mm3(qp[g], kp[g], _MT), 0.0) for g in gs]
    o_acc = [_mm3(qp[g], stp[g], _MM) + _mm3(_split(qk[g]), up[g], _MM)
             for g in gs]
    for g in gs:
        s_ref[g] = st[g] + _mm3(kp[g], up[g], _TM)

    o_all = jnp.concatenate(o_acc, axis=0)
    proj = _mm3(_split(o_all), (woh_ref[...], wol_ref[...]), _MM)
    o_ref[...] = (proj + bo_ref[...]).reshape(G, C, D)


def kernel(x, Wq, bq, Wk, bk, Wv, bv, Wbeta, bbeta, Wo, bo):
    B, T, D = x.shape
    C = _C
    G = _G
    assert T % C == 0 and B % G == 0
    full = lambda b, t: (0, 0)
    wspec = pl.BlockSpec((D, D), full)
    bspec = pl.BlockSpec((1, D), full)
    wqh, wql = _split(Wq.T)
    wkh, wkl = _split(Wk.T)
    wvh, wvl = _split(Wv.T)
    woh, wol = _split(Wo.T)
    return pl.pallas_call(
        _dn_body,
        out_shape=jax.ShapeDtypeStruct((B, T, D), x.dtype),
        grid=(B // G, T // C),
        in_specs=[
            pl.BlockSpec((G, C, D), lambda b, t: (b, t, 0)),
            wspec, wspec, wspec, wspec, wspec, wspec, wspec, wspec,
            bspec, bspec, bspec, bspec, bspec,
            pl.BlockSpec((1, 1), full),
        ],
        out_specs=pl.BlockSpec((G, C, D), lambda b, t: (b, t, 0)),
        scratch_shapes=[pltpu.VMEM((G, D, D), jnp.float32)],
        compiler_params=pltpu.CompilerParams(
            dimension_semantics=("parallel", "arbitrary"),
            vmem_limit_bytes=56 * 1024 * 1024,
        ),
        name="deltanet_chunked",
        interpret=False,
    )(x, wqh, wql, wkh, wkl, wvh, wvl, woh, wol,
      bq.reshape(1, D), bk.reshape(1, D), bv.reshape(1, D),
      bo.reshape(1, D), Wbeta.reshape(1, D), bbeta.reshape(1, 1))
